# baseline (device time: 35484 ns/iter reference)
import jax
import jax.numpy as jnp
from jax import lax
from jax.experimental import pallas as pl
from jax.experimental.pallas import tpu as pltpu

N_DEV = 16
N_TOK = 512
D_IN = 256
D_OUT = 512
E_LOCAL = 4
CHUNK = N_TOK // N_DEV


def kernel(x, router_W, route_idx, expert_W, shared_W):
    def body(x_ref, rw_ref, idx_ref, ew_ref, sw_ref, out_ref,
             partial_ref, comm_ref, send1, recv1, send2, recv2):
        my = lax.axis_index("i")

        barrier_sem = pltpu.get_barrier_semaphore()
        for d in range(1, N_DEV):
            peer = (my + d) % N_DEV
            pl.semaphore_signal(
                barrier_sem, inc=1,
                device_id=(peer,), device_id_type=pl.DeviceIdType.MESH,
            )
        pl.semaphore_wait(barrier_sem, N_DEV - 1)

        xv = x_ref[:, :]
        scores = jnp.dot(xv, rw_ref[:, :], preferred_element_type=jnp.float32)
        s_max = jnp.max(scores, axis=-1, keepdims=True)
        p = jnp.exp(scores - s_max)
        probs = p / jnp.sum(p, axis=-1, keepdims=True)
        ridx = idx_ref[:, :]
        col = lax.broadcasted_iota(jnp.int32, (N_TOK, 64), 1)
        routed = jnp.where(col == ridx, probs, 0.0)

        acc = jnp.zeros((N_TOK, D_OUT), jnp.float32)
        for e in range(E_LOCAL):
            ge = my * E_LOCAL + e
            gate = jnp.sum(jnp.where(col == ge, routed, 0.0), axis=1)
            gx = xv * gate[:, None]
            acc = acc + jnp.dot(gx, ew_ref[e], preferred_element_type=jnp.float32)
        partial_ref[...] = acc.reshape(N_DEV, CHUNK, D_OUT)

        for d in range(1, N_DEV):
            q = (my + d) % N_DEV
            pltpu.make_async_remote_copy(
                src_ref=partial_ref.at[q],
                dst_ref=comm_ref.at[my],
                send_sem=send1.at[d],
                recv_sem=recv1.at[my],
                device_id=(q,),
                device_id_type=pl.DeviceIdType.MESH,
            ).start()

        for d in range(1, N_DEV):
            s = (my + d) % N_DEV
            pltpu.make_async_remote_copy(
                src_ref=partial_ref.at[s],
                dst_ref=comm_ref.at[s],
                send_sem=send1.at[d],
                recv_sem=recv1.at[s],
                device_id=(s,),
                device_id_type=pl.DeviceIdType.MESH,
            ).wait_recv()

        slot = lax.broadcasted_iota(jnp.int32, (N_DEV, 1, 1), 0)
        gathered = jnp.where(slot != my, comm_ref[...], 0.0)
        reduced = jnp.sum(gathered, axis=0) + partial_ref[my]

        xs = x_ref[pl.ds(my * CHUNK, CHUNK), :]
        shared = jnp.dot(xs, sw_ref[:, :], preferred_element_type=jnp.float32)
        out_ref[pl.ds(my * CHUNK, CHUNK), :] = reduced + shared

        for d in range(1, N_DEV):
            q = (my + d) % N_DEV
            pltpu.make_async_remote_copy(
                src_ref=out_ref.at[pl.ds(my * CHUNK, CHUNK)],
                dst_ref=out_ref.at[pl.ds(my * CHUNK, CHUNK)],
                send_sem=send2.at[d],
                recv_sem=recv2.at[my],
                device_id=(q,),
                device_id_type=pl.DeviceIdType.MESH,
            ).start()

        for d in range(1, N_DEV):
            s = (my + d) % N_DEV
            pltpu.make_async_remote_copy(
                src_ref=out_ref.at[pl.ds(s * CHUNK, CHUNK)],
                dst_ref=out_ref.at[pl.ds(s * CHUNK, CHUNK)],
                send_sem=send2.at[d],
                recv_sem=recv2.at[s],
                device_id=(s,),
                device_id_type=pl.DeviceIdType.MESH,
            ).wait_recv()

        for d in range(1, N_DEV):
            q = (my + d) % N_DEV
            pltpu.make_async_remote_copy(
                src_ref=partial_ref.at[q],
                dst_ref=comm_ref.at[my],
                send_sem=send1.at[d],
                recv_sem=recv1.at[my],
                device_id=(q,),
                device_id_type=pl.DeviceIdType.MESH,
            ).wait_send()
            pltpu.make_async_remote_copy(
                src_ref=out_ref.at[pl.ds(my * CHUNK, CHUNK)],
                dst_ref=out_ref.at[pl.ds(my * CHUNK, CHUNK)],
                send_sem=send2.at[d],
                recv_sem=recv2.at[my],
                device_id=(q,),
                device_id_type=pl.DeviceIdType.MESH,
            ).wait_send()

    return pl.pallas_call(
        body,
        out_shape=jax.ShapeDtypeStruct((N_TOK, D_OUT), jnp.float32),
        in_specs=[pl.BlockSpec(memory_space=pltpu.VMEM)] * 5,
        out_specs=pl.BlockSpec(memory_space=pltpu.VMEM),
        scratch_shapes=[
            pltpu.VMEM((N_DEV, CHUNK, D_OUT), jnp.float32),
            pltpu.VMEM((N_DEV, CHUNK, D_OUT), jnp.float32),
            pltpu.SemaphoreType.DMA((N_DEV,)),
            pltpu.SemaphoreType.DMA((N_DEV,)),
            pltpu.SemaphoreType.DMA((N_DEV,)),
            pltpu.SemaphoreType.DMA((N_DEV,)),
        ],
        compiler_params=pltpu.CompilerParams(collective_id=0),
    )(x, router_W, route_idx, expert_W, shared_W)


# device time: 25840 ns/iter; 1.3732x vs baseline; 1.3732x over previous
import jax
import jax.numpy as jnp
from jax import lax
from jax.experimental import pallas as pl
from jax.experimental.pallas import tpu as pltpu

N_DEV = 16
N_TOK = 512
D_IN = 256
D_OUT = 512
E_LOCAL = 4
CHUNK = N_TOK // N_DEV


def kernel(x, router_W, route_idx, expert_W, shared_W):
    def body(x_ref, rw_ref, idx_ref, ew_ref, sw_ref, out_ref,
             partial_ref, comm1_ref, bcast_ref, comm2_ref,
             send1, recv1, send2, recv2):
        my = lax.axis_index("i")

        barrier_sem = pltpu.get_barrier_semaphore()
        for d in range(1, N_DEV):
            peer = (my + d) % N_DEV
            pl.semaphore_signal(
                barrier_sem, inc=1,
                device_id=(peer,), device_id_type=pl.DeviceIdType.MESH,
            )

        xv = x_ref[:, :]
        scores = jnp.dot(xv, rw_ref[:, :], preferred_element_type=jnp.float32)
        s_max = jnp.max(scores, axis=-1, keepdims=True)
        p = jnp.exp(scores - s_max)
        probs = p / jnp.sum(p, axis=-1, keepdims=True)
        ridx = idx_ref[:, :]
        col = lax.broadcasted_iota(jnp.int32, (N_TOK, 64), 1)
        routed = jnp.where(col == ridx, probs, 0.0)

        xb = xv.astype(jnp.bfloat16)
        acc = jnp.zeros((N_TOK, D_OUT), jnp.float32)
        for e in range(E_LOCAL):
            ge = my * E_LOCAL + e
            gate = jnp.sum(jnp.where(col == ge, routed, 0.0), axis=1)
            gx = xb * gate[:, None].astype(jnp.bfloat16)
            acc = acc + jnp.dot(
                gx, ew_ref[e].astype(jnp.bfloat16),
                preferred_element_type=jnp.float32,
            )
        partial_ref[...] = acc.astype(jnp.bfloat16).reshape(N_DEV, CHUNK, D_OUT)

        pl.semaphore_wait(barrier_sem, N_DEV - 1)

        for d in range(1, N_DEV):
            q = (my + d) % N_DEV
            pltpu.make_async_remote_copy(
                src_ref=partial_ref.at[q],
                dst_ref=comm1_ref.at[my],
                send_sem=send1.at[d],
                recv_sem=recv1.at[my],
                device_id=(q,),
                device_id_type=pl.DeviceIdType.MESH,
            ).start()

        xs = x_ref[pl.ds(my * CHUNK, CHUNK), :].astype(jnp.bfloat16)
        shared = jnp.dot(
            xs, sw_ref[:, :].astype(jnp.bfloat16),
            preferred_element_type=jnp.float32,
        )

        for d in range(1, N_DEV):
            s = (my + d) % N_DEV
            pltpu.make_async_remote_copy(
                src_ref=partial_ref.at[s],
                dst_ref=comm1_ref.at[s],
                send_sem=send1.at[d],
                recv_sem=recv1.at[s],
                device_id=(s,),
                device_id_type=pl.DeviceIdType.MESH,
            ).wait_recv()

        slot = lax.broadcasted_iota(jnp.int32, (N_DEV, 1, 1), 0)
        gathered = jnp.where(slot != my, comm1_ref[...].astype(jnp.float32), 0.0)
        reduced = jnp.sum(gathered, axis=0) + partial_ref[my].astype(jnp.float32)

        outc = reduced + shared
        out_ref[pl.ds(my * CHUNK, CHUNK), :] = outc
        bcast_ref[...] = outc.astype(jnp.bfloat16)

        for d in range(1, N_DEV):
            q = (my + d) % N_DEV
            pltpu.make_async_remote_copy(
                src_ref=bcast_ref,
                dst_ref=comm2_ref.at[my],
                send_sem=send2.at[d],
                recv_sem=recv2.at[my],
                device_id=(q,),
                device_id_type=pl.DeviceIdType.MESH,
            ).start()

        for d in range(1, N_DEV):
            s = (my + d) % N_DEV
            pltpu.make_async_remote_copy(
                src_ref=bcast_ref,
                dst_ref=comm2_ref.at[s],
                send_sem=send2.at[d],
                recv_sem=recv2.at[s],
                device_id=(s,),
                device_id_type=pl.DeviceIdType.MESH,
            ).wait_recv()
            out_ref[pl.ds(s * CHUNK, CHUNK), :] = comm2_ref[s].astype(jnp.float32)

        for d in range(1, N_DEV):
            q = (my + d) % N_DEV
            pltpu.make_async_remote_copy(
                src_ref=partial_ref.at[q],
                dst_ref=comm1_ref.at[my],
                send_sem=send1.at[d],
                recv_sem=recv1.at[my],
                device_id=(q,),
                device_id_type=pl.DeviceIdType.MESH,
            ).wait_send()
            pltpu.make_async_remote_copy(
                src_ref=bcast_ref,
                dst_ref=comm2_ref.at[my],
                send_sem=send2.at[d],
                recv_sem=recv2.at[my],
                device_id=(q,),
                device_id_type=pl.DeviceIdType.MESH,
            ).wait_send()

    return pl.pallas_call(
        body,
        out_shape=jax.ShapeDtypeStruct((N_TOK, D_OUT), jnp.float32),
        in_specs=[pl.BlockSpec(memory_space=pltpu.VMEM)] * 5,
        out_specs=pl.BlockSpec(memory_space=pltpu.VMEM),
        scratch_shapes=[
            pltpu.VMEM((N_DEV, CHUNK, D_OUT), jnp.bfloat16),
            pltpu.VMEM((N_DEV, CHUNK, D_OUT), jnp.bfloat16),
            pltpu.VMEM((CHUNK, D_OUT), jnp.bfloat16),
            pltpu.VMEM((N_DEV, CHUNK, D_OUT), jnp.bfloat16),
            pltpu.SemaphoreType.DMA((N_DEV,)),
            pltpu.SemaphoreType.DMA((N_DEV,)),
            pltpu.SemaphoreType.DMA((N_DEV,)),
            pltpu.SemaphoreType.DMA((N_DEV,)),
        ],
        compiler_params=pltpu.CompilerParams(collective_id=0),
    )(x, router_W, route_idx, expert_W, shared_W)


# device time: 25052 ns/iter; 1.4164x vs baseline; 1.0315x over previous
import jax
import jax.numpy as jnp
from jax import lax
from jax.experimental import pallas as pl
from jax.experimental.pallas import tpu as pltpu

N_DEV = 16
N_TOK = 512
D_IN = 256
D_OUT = 512
E_LOCAL = 4
CHUNK = N_TOK // N_DEV


def kernel(x, router_W, route_idx, expert_W, shared_W):
    def body(x_ref, rw_ref, idx_ref, ew_ref, sw_ref, out_ref,
             partial_ref, comm1_ref, bcast_ref, comm2_ref,
             send1, recv1, send2, recv2, ready):
        my = lax.axis_index("i")

        for d in range(1, N_DEV):
            peer = (my + d) % N_DEV
            pl.semaphore_signal(
                ready.at[my], inc=1,
                device_id=(peer,), device_id_type=pl.DeviceIdType.MESH,
            )

        barrier_sem = pltpu.get_barrier_semaphore()
        pl.semaphore_signal(barrier_sem, inc=1)
        pl.semaphore_wait(barrier_sem, 1)

        xv = x_ref[:, :]
        scores = jnp.dot(xv, rw_ref[:, :], preferred_element_type=jnp.float32)
        s_max = jnp.max(scores, axis=-1, keepdims=True)
        p = jnp.exp(scores - s_max)
        probs = p / jnp.sum(p, axis=-1, keepdims=True)
        ridx = idx_ref[:, :]
        col = lax.broadcasted_iota(jnp.int32, (N_TOK, 64), 1)
        routed = jnp.where(col == ridx, probs, 0.0)

        xb = xv.astype(jnp.bfloat16)
        acc = jnp.zeros((N_TOK, D_OUT), jnp.float32)
        for e in range(E_LOCAL):
            ge = my * E_LOCAL + e
            gate = jnp.sum(jnp.where(col == ge, routed, 0.0), axis=1)
            gx = xb * gate[:, None].astype(jnp.bfloat16)
            acc = acc + jnp.dot(
                gx, ew_ref[e].astype(jnp.bfloat16),
                preferred_element_type=jnp.float32,
            )
        partial_ref[...] = acc.astype(jnp.bfloat16).reshape(N_DEV, CHUNK, D_OUT)

        for d in range(1, N_DEV):
            q = (my + d) % N_DEV
            pl.semaphore_wait(ready.at[q], 1)
            pltpu.make_async_remote_copy(
                src_ref=partial_ref.at[q],
                dst_ref=comm1_ref.at[my],
                send_sem=send1.at[d],
                recv_sem=recv1.at[my],
                device_id=(q,),
                device_id_type=pl.DeviceIdType.MESH,
            ).start()

        xs = x_ref[pl.ds(my * CHUNK, CHUNK), :].astype(jnp.bfloat16)
        shared = jnp.dot(
            xs, sw_ref[:, :].astype(jnp.bfloat16),
            preferred_element_type=jnp.float32,
        )
        reduced = shared + partial_ref[my].astype(jnp.float32)

        for d in range(1, N_DEV):
            s = (my + d) % N_DEV
            pltpu.make_async_remote_copy(
                src_ref=partial_ref.at[s],
                dst_ref=comm1_ref.at[s],
                send_sem=send1.at[d],
                recv_sem=recv1.at[s],
                device_id=(s,),
                device_id_type=pl.DeviceIdType.MESH,
            ).wait_recv()
            reduced = reduced + comm1_ref[s].astype(jnp.float32)

        bcast_ref[...] = reduced.astype(jnp.bfloat16)

        for d in range(1, N_DEV):
            q = (my + d) % N_DEV
            pltpu.make_async_remote_copy(
                src_ref=bcast_ref,
                dst_ref=comm2_ref.at[my],
                send_sem=send2.at[d],
                recv_sem=recv2.at[my],
                device_id=(q,),
                device_id_type=pl.DeviceIdType.MESH,
            ).start()

        for d in range(1, N_DEV):
            s = (my + d) % N_DEV
            pltpu.make_async_remote_copy(
                src_ref=bcast_ref,
                dst_ref=comm2_ref.at[s],
                send_sem=send2.at[d],
                recv_sem=recv2.at[s],
                device_id=(s,),
                device_id_type=pl.DeviceIdType.MESH,
            ).wait_recv()

        out_ref[...] = comm2_ref[...].astype(jnp.float32).reshape(N_TOK, D_OUT)
        out_ref[pl.ds(my * CHUNK, CHUNK), :] = reduced

        for d in range(1, N_DEV):
            q = (my + d) % N_DEV
            pltpu.make_async_remote_copy(
                src_ref=partial_ref.at[q],
                dst_ref=comm1_ref.at[my],
                send_sem=send1.at[d],
                recv_sem=recv1.at[my],
                device_id=(q,),
                device_id_type=pl.DeviceIdType.MESH,
            ).wait_send()
            pltpu.make_async_remote_copy(
                src_ref=bcast_ref,
                dst_ref=comm2_ref.at[my],
                send_sem=send2.at[d],
                recv_sem=recv2.at[my],
                device_id=(q,),
                device_id_type=pl.DeviceIdType.MESH,
            ).wait_send()

    return pl.pallas_call(
        body,
        out_shape=jax.ShapeDtypeStruct((N_TOK, D_OUT), jnp.float32),
        in_specs=[pl.BlockSpec(memory_space=pltpu.VMEM)] * 5,
        out_specs=pl.BlockSpec(memory_space=pltpu.VMEM),
        scratch_shapes=[
            pltpu.VMEM((N_DEV, CHUNK, D_OUT), jnp.bfloat16),
            pltpu.VMEM((N_DEV, CHUNK, D_OUT), jnp.bfloat16),
            pltpu.VMEM((CHUNK, D_OUT), jnp.bfloat16),
            pltpu.VMEM((N_DEV, CHUNK, D_OUT), jnp.bfloat16),
            pltpu.SemaphoreType.DMA((N_DEV,)),
            pltpu.SemaphoreType.DMA((N_DEV,)),
            pltpu.SemaphoreType.DMA((N_DEV,)),
            pltpu.SemaphoreType.DMA((N_DEV,)),
            pltpu.SemaphoreType.REGULAR((N_DEV,)),
        ],
        compiler_params=pltpu.CompilerParams(collective_id=0),
    )(x, router_W, route_idx, expert_W, shared_W)
